# baseline (device time: 9179 ns/iter reference)
import jax
import jax.numpy as jnp
from jax import lax
from jax.experimental import pallas as pl
from jax.experimental.pallas import tpu as pltpu

N_Z = 4


def kernel(x, dy, gamma):
    m, d = x.shape

    def body(x_ref, dy_ref, out_ref, comm_ref, send_sems, recv_sems):
        my_x = lax.axis_index("x")
        my_y = lax.axis_index("y")
        my_z = lax.axis_index("z")

        xv = x_ref[:, :]
        dyv = dy_ref[:, :]
        mu = jnp.mean(xv, axis=1, keepdims=True)
        xc = xv - mu
        var = jnp.mean(xc * xc, axis=1, keepdims=True)
        rstd = lax.rsqrt(var + 1e-5)
        xhat = xc * rstd
        dgamma = jnp.sum(dyv * xhat, axis=0, keepdims=True)
        dbeta = jnp.sum(dyv, axis=0, keepdims=True)
        comm_ref[pl.ds(my_z, 1), :, :] = jnp.concatenate(
            [dgamma, dbeta], axis=0
        )[None]

        barrier_sem = pltpu.get_barrier_semaphore()
        for dz in range(1, N_Z):
            peer_z = lax.rem(my_z + dz, N_Z)
            pl.semaphore_signal(
                barrier_sem,
                inc=1,
                device_id=(my_x, my_y, peer_z),
                device_id_type=pl.DeviceIdType.MESH,
            )
        pl.semaphore_wait(barrier_sem, N_Z - 1)

        sends = []
        for dz in range(1, N_Z):
            peer_z = lax.rem(my_z + dz, N_Z)
            rdma = pltpu.make_async_remote_copy(
                src_ref=comm_ref.at[my_z],
                dst_ref=comm_ref.at[my_z],
                send_sem=send_sems.at[dz - 1],
                recv_sem=recv_sems.at[my_z],
                device_id=(my_x, my_y, peer_z),
                device_id_type=pl.DeviceIdType.MESH,
            )
            rdma.start()
            sends.append(rdma)

        for dz in range(1, N_Z):
            src_z = lax.rem(my_z + dz, N_Z)
            recv = pltpu.make_async_remote_copy(
                src_ref=comm_ref.at[src_z],
                dst_ref=comm_ref.at[src_z],
                send_sem=send_sems.at[dz - 1],
                recv_sem=recv_sems.at[src_z],
                device_id=(my_x, my_y, src_z),
                device_id_type=pl.DeviceIdType.MESH,
            )
            recv.wait_recv()

        for s in sends:
            s.wait_send()

        out_ref[:, :] = (
            comm_ref[0] + comm_ref[1] + comm_ref[2] + comm_ref[3]
        )

    return pl.pallas_call(
        body,
        out_shape=jax.ShapeDtypeStruct((2, d), jnp.float32),
        in_specs=[
            pl.BlockSpec(memory_space=pltpu.VMEM),
            pl.BlockSpec(memory_space=pltpu.VMEM),
        ],
        out_specs=pl.BlockSpec(memory_space=pltpu.VMEM),
        scratch_shapes=[
            pltpu.VMEM((N_Z, 2, d), jnp.float32),
            pltpu.SemaphoreType.DMA((N_Z - 1,)),
            pltpu.SemaphoreType.DMA((N_Z,)),
        ],
        compiler_params=pltpu.CompilerParams(collective_id=0),
    )(x, dy)


# device time: 8772 ns/iter; 1.0464x vs baseline; 1.0464x over previous
import jax
import jax.numpy as jnp
from jax import lax
from jax.experimental import pallas as pl
from jax.experimental.pallas import tpu as pltpu

N_Z = 4


def kernel(x, dy, gamma):
    m, d = x.shape

    def body(x_ref, dy_ref, out_ref, comm_ref, send_sems, recv_sems):
        my_x = lax.axis_index("x")
        my_y = lax.axis_index("y")
        my_z = lax.axis_index("z")

        barrier_sem = pltpu.get_barrier_semaphore()
        for dz in range(1, N_Z):
            peer_z = lax.rem(my_z + dz, N_Z)
            pl.semaphore_signal(
                barrier_sem,
                inc=1,
                device_id=(my_x, my_y, peer_z),
                device_id_type=pl.DeviceIdType.MESH,
            )

        xv = x_ref[:, :]
        dyv = dy_ref[:, :]
        mu = jnp.mean(xv, axis=1, keepdims=True)
        var = jnp.mean(xv * xv, axis=1, keepdims=True) - mu * mu
        rstd = lax.rsqrt(var + 1e-5)
        xhat = (xv - mu) * rstd
        dgamma = jnp.sum(dyv * xhat, axis=0, keepdims=True)
        dbeta = jnp.sum(dyv, axis=0, keepdims=True)
        comm_ref[pl.ds(my_z, 1), :, :] = jnp.concatenate(
            [dgamma, dbeta], axis=0
        )[None]

        pl.semaphore_wait(barrier_sem, N_Z - 1)

        sends = []
        for dz in range(1, N_Z):
            peer_z = lax.rem(my_z + dz, N_Z)
            rdma = pltpu.make_async_remote_copy(
                src_ref=comm_ref.at[my_z],
                dst_ref=comm_ref.at[my_z],
                send_sem=send_sems.at[dz - 1],
                recv_sem=recv_sems.at[my_z],
                device_id=(my_x, my_y, peer_z),
                device_id_type=pl.DeviceIdType.MESH,
            )
            rdma.start()
            sends.append(rdma)

        for dz in range(1, N_Z):
            src_z = lax.rem(my_z + dz, N_Z)
            recv = pltpu.make_async_remote_copy(
                src_ref=comm_ref.at[src_z],
                dst_ref=comm_ref.at[src_z],
                send_sem=send_sems.at[dz - 1],
                recv_sem=recv_sems.at[src_z],
                device_id=(my_x, my_y, src_z),
                device_id_type=pl.DeviceIdType.MESH,
            )
            recv.wait_recv()

        for s in sends:
            s.wait_send()

        out_ref[:, :] = (
            comm_ref[0] + comm_ref[1] + comm_ref[2] + comm_ref[3]
        )

    return pl.pallas_call(
        body,
        out_shape=jax.ShapeDtypeStruct((2, d), jnp.float32),
        in_specs=[
            pl.BlockSpec(memory_space=pltpu.VMEM),
            pl.BlockSpec(memory_space=pltpu.VMEM),
        ],
        out_specs=pl.BlockSpec(memory_space=pltpu.VMEM),
        scratch_shapes=[
            pltpu.VMEM((N_Z, 2, d), jnp.float32),
            pltpu.SemaphoreType.DMA((N_Z - 1,)),
            pltpu.SemaphoreType.DMA((N_Z,)),
        ],
        compiler_params=pltpu.CompilerParams(collective_id=0),
    )(x, dy)
